# Initial kernel scaffold; baseline (speedup 1.0000x reference)
import functools

import jax
import jax.numpy as jnp
from jax import lax
from jax.experimental import pallas as pl
from jax.experimental.pallas import tpu as pltpu
from jax.experimental.pallas import tpu_sc as plsc

N_NODES = 10000
N_FEAT = 128
N_EDGES = 320000

NUM_CORES = 2
NUM_SUBCORES = 16
NUM_WORKERS = NUM_CORES * NUM_SUBCORES
EDGES_PER_WORKER = N_EDGES // NUM_WORKERS
CHUNK = 80
N_CHUNKS = EDGES_PER_WORKER // CHUNK
GROUPS = CHUNK // 16


def _scale_body(h_ref, r_ref, o_ref):
    o_ref[...] = h_ref[...] * r_ref[...]


def _scale_h_by_r(h, r):
    return pl.pallas_call(
        _scale_body,
        out_shape=jax.ShapeDtypeStruct((N_NODES, N_FEAT), jnp.float32),
    )(h, r.reshape(1, N_FEAT))


_mesh = plsc.VectorSubcoreMesh(core_axis_name="c", subcore_axis_name="s")


@functools.partial(
    pl.kernel,
    mesh=_mesh,
    out_type=jax.ShapeDtypeStruct((N_EDGES,), jnp.float32),
    scratch_types=[
        pltpu.VMEM((CHUNK,), jnp.int32),
        pltpu.VMEM((CHUNK,), jnp.int32),
        pltpu.VMEM((CHUNK, N_FEAT), jnp.float32),
        pltpu.VMEM((CHUNK, N_FEAT), jnp.float32),
        pltpu.VMEM((CHUNK,), jnp.float32),
        pltpu.SemaphoreType.DMA,
        pltpu.SemaphoreType.DMA,
    ],
)
def _edge_scores(hr_hbm, h_hbm, ei_hbm, out_hbm,
                 idx_u, idx_v, u_rows, v_rows, out_buf, sem_u, sem_v):
    wid = lax.axis_index("s") * NUM_CORES + lax.axis_index("c")
    base0 = wid * EDGES_PER_WORKER
    lanes = lax.iota(jnp.int32, 16)

    def chunk_body(c, carry):
        base = base0 + c * CHUNK
        pltpu.sync_copy(ei_hbm.at[0, pl.ds(base, CHUNK)], idx_u)
        pltpu.sync_copy(ei_hbm.at[1, pl.ds(base, CHUNK)], idx_v)
        cp_u = pltpu.async_copy(hr_hbm.at[idx_u], u_rows, sem_u)
        cp_v = pltpu.async_copy(h_hbm.at[idx_v], v_rows, sem_v)
        cp_u.wait()
        cp_v.wait()

        def group_body(g, gcarry):
            e_lanes = lanes + g * 16
            acc = jnp.zeros((16,), jnp.float32)
            for f in range(N_FEAT):
                fv = jnp.full((16,), f, jnp.int32)
                u = plsc.load_gather(u_rows, [e_lanes, fv])
                v = plsc.load_gather(v_rows, [e_lanes, fv])
                acc = acc + u * v
            out_buf[pl.ds(g * 16, 16)] = acc
            return gcarry

        lax.fori_loop(0, GROUPS, group_body, 0)
        pltpu.sync_copy(out_buf, out_hbm.at[pl.ds(base, CHUNK)])
        return carry

    lax.fori_loop(0, N_CHUNKS, chunk_body, 0)


def kernel(h, edge_index, r):
    hr = _scale_h_by_r(h, r)
    return _edge_scores(hr, h, edge_index)


# speedup vs baseline: 2.4559x; 2.4559x over previous
import functools

import jax
import jax.numpy as jnp
from jax import lax
from jax.experimental import pallas as pl
from jax.experimental.pallas import tpu as pltpu
from jax.experimental.pallas import tpu_sc as plsc

N_NODES = 10000
N_FEAT = 128
N_EDGES = 320000

NUM_CORES = 2
NUM_SUBCORES = 16
NUM_WORKERS = NUM_CORES * NUM_SUBCORES
EDGES_PER_WORKER = N_EDGES // NUM_WORKERS
CHUNK = 80
N_CHUNKS = EDGES_PER_WORKER // CHUNK
GROUPS = CHUNK // 16
PART_PITCH = 17


def _scale_body(h_ref, r_ref, o_ref):
    o_ref[...] = h_ref[...] * r_ref[...]


def _scale_h_by_r(h, r):
    return pl.pallas_call(
        _scale_body,
        out_shape=jax.ShapeDtypeStruct((N_NODES, N_FEAT), jnp.float32),
    )(h, r.reshape(1, N_FEAT))


_mesh = plsc.VectorSubcoreMesh(core_axis_name="c", subcore_axis_name="s")


@functools.partial(
    pl.kernel,
    mesh=_mesh,
    out_type=jax.ShapeDtypeStruct((N_EDGES,), jnp.float32),
    scratch_types=[
        pltpu.VMEM((CHUNK,), jnp.int32),
        pltpu.VMEM((CHUNK,), jnp.int32),
        pltpu.VMEM((CHUNK, N_FEAT), jnp.float32),
        pltpu.VMEM((CHUNK, N_FEAT), jnp.float32),
        pltpu.VMEM((CHUNK,), jnp.float32),
        pltpu.SemaphoreType.DMA,
        pltpu.SemaphoreType.DMA,
    ],
)
def _edge_scores(hr_hbm, h_hbm, ei_hbm, out_hbm,
                 idx_u, idx_v, u_rows, v_rows, out_buf, sem_u, sem_v):
    wid = lax.axis_index("s") * NUM_CORES + lax.axis_index("c")
    base0 = wid * EDGES_PER_WORKER
    lanes = lax.iota(jnp.int32, 16)

    def chunk_body(c, carry):
        base = base0 + c * CHUNK
        pltpu.sync_copy(ei_hbm.at[pl.ds(base, CHUNK)], idx_u)
        pltpu.sync_copy(ei_hbm.at[pl.ds(N_EDGES + base, CHUNK)], idx_v)
        cp_u = pltpu.async_copy(hr_hbm.at[idx_u], u_rows, sem_u)
        cp_v = pltpu.async_copy(h_hbm.at[idx_v], v_rows, sem_v)
        cp_u.wait()
        cp_v.wait()

        def group_body(g, gcarry):
            scores = jnp.zeros((16,), jnp.float32)
            for t in range(16):
                e = g * 16 + t
                p = [u_rows[e, pl.ds(16 * j, 16)] * v_rows[e, pl.ds(16 * j, 16)]
                     for j in range(N_FEAT // 16)]
                acc = ((p[0] + p[1]) + (p[2] + p[3])) + ((p[4] + p[5]) + (p[6] + p[7]))
                for sh in (8, 4, 2, 1):
                    acc = acc + jnp.take(acc, (lanes + sh) & 15)
                scores = jnp.where(lanes == t, acc, scores)
            out_buf[pl.ds(g * 16, 16)] = scores
            return gcarry

        lax.fori_loop(0, GROUPS, group_body, 0)
        pltpu.sync_copy(out_buf, out_hbm.at[pl.ds(base, CHUNK)])
        return carry

    lax.fori_loop(0, N_CHUNKS, chunk_body, 0)


def kernel(h, edge_index, r):
    hr = _scale_h_by_r(h, r)
    return _edge_scores(hr, h, edge_index.reshape(-1))


# speedup vs baseline: 4.4274x; 1.8028x over previous
import functools

import jax
import jax.numpy as jnp
from jax import lax
from jax.experimental import pallas as pl
from jax.experimental.pallas import tpu as pltpu
from jax.experimental.pallas import tpu_sc as plsc

N_NODES = 10000
N_FEAT = 128
N_EDGES = 320000

NUM_CORES = 2
NUM_SUBCORES = 16
NUM_WORKERS = NUM_CORES * NUM_SUBCORES
EDGES_PER_WORKER = N_EDGES // NUM_WORKERS
CHUNK = 128
GROUPS = CHUNK // 16
N_CHUNKS = -(-EDGES_PER_WORKER // CHUNK)
N_PAIRS = N_CHUNKS // 2
LAST_BASE = EDGES_PER_WORKER - CHUNK


def _scale_body(h_ref, r_ref, o_ref):
    o_ref[...] = h_ref[...] * r_ref[...]


def _scale_h_by_r(h, r):
    return pl.pallas_call(
        _scale_body,
        out_shape=jax.ShapeDtypeStruct((N_NODES, N_FEAT), jnp.float32),
    )(h, r.reshape(1, N_FEAT))


_mesh = plsc.VectorSubcoreMesh(core_axis_name="c", subcore_axis_name="s")


@functools.partial(
    pl.kernel,
    mesh=_mesh,
    out_type=jax.ShapeDtypeStruct((N_EDGES,), jnp.float32),
    scratch_types=[
        pltpu.VMEM((EDGES_PER_WORKER,), jnp.int32),
        pltpu.VMEM((EDGES_PER_WORKER,), jnp.int32),
        pltpu.VMEM((CHUNK, N_FEAT), jnp.float32),
        pltpu.VMEM((CHUNK, N_FEAT), jnp.float32),
        pltpu.VMEM((CHUNK, N_FEAT), jnp.float32),
        pltpu.VMEM((CHUNK, N_FEAT), jnp.float32),
        pltpu.VMEM((CHUNK,), jnp.float32),
        pltpu.VMEM((CHUNK,), jnp.float32),
        pltpu.SemaphoreType.DMA,
        pltpu.SemaphoreType.DMA,
        pltpu.SemaphoreType.DMA,
        pltpu.SemaphoreType.DMA,
        pltpu.SemaphoreType.DMA,
        pltpu.SemaphoreType.DMA,
    ],
)
def _edge_scores(hr_hbm, h_hbm, ei_hbm, out_hbm,
                 idx_u, idx_v, u0, u1, v0, v1, out0, out1,
                 sem_u0, sem_u1, sem_v0, sem_v1, sem_o0, sem_o1):
    wid = lax.axis_index("s") * NUM_CORES + lax.axis_index("c")
    base0 = wid * EDGES_PER_WORKER
    lanes = lax.iota(jnp.int32, 16)

    pltpu.sync_copy(ei_hbm.at[pl.ds(base0, EDGES_PER_WORKER)], idx_u)
    pltpu.sync_copy(ei_hbm.at[pl.ds(N_EDGES + base0, EDGES_PER_WORKER)], idx_v)

    def gathers(base, u_buf, v_buf, su, sv):
        cu = pltpu.async_copy(hr_hbm.at[idx_u.at[pl.ds(base, CHUNK)]], u_buf, su)
        cv = pltpu.async_copy(h_hbm.at[idx_v.at[pl.ds(base, CHUNK)]], v_buf, sv)
        return cu, cv

    def compute(u_buf, v_buf, out_buf):
        def group_body(g, gcarry):
            scores = jnp.zeros((16,), jnp.float32)
            for t in range(16):
                e = g * 16 + t
                p = [u_buf[e, pl.ds(16 * j, 16)] * v_buf[e, pl.ds(16 * j, 16)]
                     for j in range(N_FEAT // 16)]
                acc = ((p[0] + p[1]) + (p[2] + p[3])) + ((p[4] + p[5]) + (p[6] + p[7]))
                for sh in (8, 4, 2, 1):
                    acc = acc + jnp.take(acc, (lanes + sh) & 15)
                scores = jnp.where(lanes == t, acc, scores)
            out_buf[pl.ds(g * 16, 16)] = scores
            return gcarry

        lax.fori_loop(0, GROUPS, group_body, 0)

    def writeback(base, out_buf, sem):
        return pltpu.async_copy(out_buf, out_hbm.at[pl.ds(base0 + base, CHUNK)], sem)

    def reclaim(out_buf, sem):
        pltpu.make_async_copy(out_buf, out_hbm.at[pl.ds(base0, CHUNK)], sem).wait()

    cu, cv = gathers(0, u0, v0, sem_u0, sem_v0)
    cu.wait()
    cv.wait()

    def pair_body(i, carry):
        c0_base = (2 * i) * CHUNK
        c1_base = c0_base + CHUNK
        n0_base = jnp.minimum(c0_base + 2 * CHUNK, LAST_BASE)

        cu1, cv1 = gathers(c1_base, u1, v1, sem_u1, sem_v1)

        @pl.when(i > 0)
        def _():
            reclaim(out0, sem_o0)

        compute(u0, v0, out0)
        writeback(c0_base, out0, sem_o0)
        cu1.wait()
        cv1.wait()

        cu0, cv0 = gathers(n0_base, u0, v0, sem_u0, sem_v0)

        @pl.when(i > 0)
        def _():
            reclaim(out1, sem_o1)

        compute(u1, v1, out1)
        writeback(c1_base, out1, sem_o1)
        cu0.wait()
        cv0.wait()
        return carry

    lax.fori_loop(0, N_PAIRS, pair_body, 0)

    reclaim(out0, sem_o0)
    compute(u0, v0, out0)
    cp = writeback(LAST_BASE, out0, sem_o0)
    reclaim(out1, sem_o1)
    cp.wait()


def kernel(h, edge_index, r):
    hr = _scale_h_by_r(h, r)
    return _edge_scores(hr, h, edge_index.reshape(-1))


# speedup vs baseline: 4.6173x; 1.0429x over previous
import functools

import jax
import jax.numpy as jnp
from jax import lax
from jax.experimental import pallas as pl
from jax.experimental.pallas import tpu as pltpu
from jax.experimental.pallas import tpu_sc as plsc

N_NODES = 10000
N_FEAT = 128
N_EDGES = 320000

NUM_CORES = 2
NUM_SUBCORES = 16
NUM_WORKERS = NUM_CORES * NUM_SUBCORES
EDGES_PER_WORKER = N_EDGES // NUM_WORKERS
CHUNK = 128
GROUPS = CHUNK // 16
N_CHUNKS = -(-EDGES_PER_WORKER // CHUNK)
N_PAIRS = N_CHUNKS // 2
LAST_BASE = EDGES_PER_WORKER - CHUNK


def _scale_body(h_ref, r_ref, o_ref):
    o_ref[...] = h_ref[...] * r_ref[...]


def _scale_h_by_r(h, r):
    return pl.pallas_call(
        _scale_body,
        out_shape=jax.ShapeDtypeStruct((N_NODES, N_FEAT), jnp.float32),
    )(h, r.reshape(1, N_FEAT))


_mesh = plsc.VectorSubcoreMesh(core_axis_name="c", subcore_axis_name="s")


@functools.partial(
    pl.kernel,
    mesh=_mesh,
    out_type=jax.ShapeDtypeStruct((N_EDGES,), jnp.float32),
    scratch_types=[
        pltpu.VMEM((EDGES_PER_WORKER,), jnp.int32),
        pltpu.VMEM((EDGES_PER_WORKER,), jnp.int32),
        pltpu.VMEM((CHUNK, N_FEAT), jnp.float32),
        pltpu.VMEM((CHUNK, N_FEAT), jnp.float32),
        pltpu.VMEM((CHUNK, N_FEAT), jnp.float32),
        pltpu.VMEM((CHUNK, N_FEAT), jnp.float32),
        pltpu.VMEM((CHUNK,), jnp.float32),
        pltpu.VMEM((CHUNK,), jnp.float32),
        pltpu.SemaphoreType.DMA,
        pltpu.SemaphoreType.DMA,
        pltpu.SemaphoreType.DMA,
        pltpu.SemaphoreType.DMA,
        pltpu.SemaphoreType.DMA,
        pltpu.SemaphoreType.DMA,
    ],
)
def _edge_scores(hr_hbm, h_hbm, ei_hbm, out_hbm,
                 idx_u, idx_v, u0, u1, v0, v1, out0, out1,
                 sem_u0, sem_u1, sem_v0, sem_v1, sem_o0, sem_o1):
    wid = lax.axis_index("s") * NUM_CORES + lax.axis_index("c")
    base0 = wid * EDGES_PER_WORKER
    lanes = lax.iota(jnp.int32, 16)

    pltpu.sync_copy(ei_hbm.at[pl.ds(base0, EDGES_PER_WORKER)], idx_u)
    pltpu.sync_copy(ei_hbm.at[pl.ds(N_EDGES + base0, EDGES_PER_WORKER)], idx_v)

    def gathers(base, u_buf, v_buf, su, sv):
        cu = pltpu.async_copy(hr_hbm.at[idx_u.at[pl.ds(base, CHUNK)]], u_buf, su)
        cv = pltpu.async_copy(h_hbm.at[idx_v.at[pl.ds(base, CHUNK)]], v_buf, sv)
        return cu, cv

    masks = (lanes < 8, (lanes & 4) == 0, (lanes & 2) == 0, (lanes & 1) == 0)
    shifts = (8, 4, 2, 1)
    bitrev = ((lanes & 1) * 8) + (((lanes >> 1) & 1) * 4) \
        + (((lanes >> 2) & 1) * 2) + ((lanes >> 3) & 1)

    def step(v, sh):
        return v + jnp.take(v, lanes ^ sh)

    def compute(u_buf, v_buf, out_buf):
        def group_body(g, gcarry):
            stack = []
            for t in range(16):
                e = g * 16 + t
                p = [u_buf[e, pl.ds(16 * j, 16)] * v_buf[e, pl.ds(16 * j, 16)]
                     for j in range(N_FEAT // 16)]
                cur = ((p[0] + p[1]) + (p[2] + p[3])) + ((p[4] + p[5]) + (p[6] + p[7]))
                level = 0
                while stack and stack[-1][0] == level:
                    prev = stack.pop()[1]
                    cur = jnp.where(masks[level],
                                    step(prev, shifts[level]),
                                    step(cur, shifts[level]))
                    level += 1
                stack.append((level, cur))
            scores = jnp.take(stack[-1][1], bitrev)
            out_buf[pl.ds(g * 16, 16)] = scores
            return gcarry

        lax.fori_loop(0, GROUPS, group_body, 0)

    def writeback(base, out_buf, sem):
        return pltpu.async_copy(out_buf, out_hbm.at[pl.ds(base0 + base, CHUNK)], sem)

    def reclaim(out_buf, sem):
        pltpu.make_async_copy(out_buf, out_hbm.at[pl.ds(base0, CHUNK)], sem).wait()

    cu, cv = gathers(0, u0, v0, sem_u0, sem_v0)
    cu.wait()
    cv.wait()

    def pair_body(i, carry):
        c0_base = (2 * i) * CHUNK
        c1_base = c0_base + CHUNK
        n0_base = jnp.minimum(c0_base + 2 * CHUNK, LAST_BASE)

        cu1, cv1 = gathers(c1_base, u1, v1, sem_u1, sem_v1)

        @pl.when(i > 0)
        def _():
            reclaim(out0, sem_o0)

        compute(u0, v0, out0)
        writeback(c0_base, out0, sem_o0)
        cu1.wait()
        cv1.wait()

        cu0, cv0 = gathers(n0_base, u0, v0, sem_u0, sem_v0)

        @pl.when(i > 0)
        def _():
            reclaim(out1, sem_o1)

        compute(u1, v1, out1)
        writeback(c1_base, out1, sem_o1)
        cu0.wait()
        cv0.wait()
        return carry

    lax.fori_loop(0, N_PAIRS, pair_body, 0)

    reclaim(out0, sem_o0)
    compute(u0, v0, out0)
    cp = writeback(LAST_BASE, out0, sem_o0)
    reclaim(out1, sem_o1)
    cp.wait()


def kernel(h, edge_index, r):
    hr = _scale_h_by_r(h, r)
    return _edge_scores(hr, h, edge_index.reshape(-1))


# speedup vs baseline: 8.9197x; 1.9318x over previous
import functools

import jax
import jax.numpy as jnp
from jax import lax
from jax.experimental import pallas as pl
from jax.experimental.pallas import tpu as pltpu
from jax.experimental.pallas import tpu_sc as plsc

N_NODES = 10000
N_FEAT = 128
N_EDGES = 320000

NUM_CORES = 2
NUM_SUBCORES = 16
NUM_WORKERS = NUM_CORES * NUM_SUBCORES
EDGES_PER_WORKER = N_EDGES // NUM_WORKERS
CHUNK = 128
GROUPS = CHUNK // 16
N_CHUNKS = -(-EDGES_PER_WORKER // CHUNK)
N_PAIRS = N_CHUNKS // 2
LAST_BASE = EDGES_PER_WORKER - CHUNK


def _scale_body(h_ref, r_ref, o_ref):
    o_ref[...] = h_ref[...] * r_ref[...]


def _scale_h_by_r(h, r):
    return pl.pallas_call(
        _scale_body,
        out_shape=jax.ShapeDtypeStruct((N_NODES, N_FEAT), jnp.float32),
    )(h, r.reshape(1, N_FEAT))


_mesh = plsc.VectorSubcoreMesh(core_axis_name="c", subcore_axis_name="s")


@functools.partial(
    pl.kernel,
    mesh=_mesh,
    out_type=jax.ShapeDtypeStruct((N_EDGES,), jnp.float32),
    scratch_types=[
        pltpu.VMEM((EDGES_PER_WORKER,), jnp.int32),
        pltpu.VMEM((EDGES_PER_WORKER,), jnp.int32),
        pltpu.VMEM((CHUNK, N_FEAT), jnp.float32),
        pltpu.VMEM((CHUNK, N_FEAT), jnp.float32),
        pltpu.VMEM((CHUNK, N_FEAT), jnp.float32),
        pltpu.VMEM((CHUNK, N_FEAT), jnp.float32),
        pltpu.VMEM((CHUNK,), jnp.float32),
        pltpu.VMEM((CHUNK,), jnp.float32),
        pltpu.SemaphoreType.DMA,
        pltpu.SemaphoreType.DMA,
        pltpu.SemaphoreType.DMA,
        pltpu.SemaphoreType.DMA,
        pltpu.SemaphoreType.DMA,
        pltpu.SemaphoreType.DMA,
    ],
)
def _edge_scores(hr_hbm, h_hbm, ei_hbm, out_hbm,
                 idx_u, idx_v, u0, u1, v0, v1, out0, out1,
                 sem_u0, sem_u1, sem_v0, sem_v1, sem_o0, sem_o1):
    wid = lax.axis_index("s") * NUM_CORES + lax.axis_index("c")
    base0 = wid * EDGES_PER_WORKER
    lanes = lax.iota(jnp.int32, 16)

    pltpu.sync_copy(ei_hbm.at[pl.ds(base0, EDGES_PER_WORKER)], idx_u)
    pltpu.sync_copy(ei_hbm.at[pl.ds(N_EDGES + base0, EDGES_PER_WORKER)], idx_v)

    def gathers(base, u_buf, v_buf, su, sv):
        cu = pltpu.async_copy(hr_hbm.at[idx_u.at[pl.ds(base, CHUNK)]], u_buf, su)
        cv = pltpu.async_copy(h_hbm.at[idx_v.at[pl.ds(base, CHUNK)]], v_buf, sv)
        return cu, cv

    msk8 = lanes < 8
    msk4 = (lanes & 4) == 0
    quad_pat = (lanes & 1) * 8 + ((lanes >> 1) & 1) * 4

    def step(v, sh):
        return v + jnp.take(v, lanes ^ sh)

    def compute(u_buf, v_buf, out_buf):
        def quad_body(k, scores):
            q = k & 3
            accs = []
            for t in range(4):
                e = k * 4 + t
                p = [u_buf[e, pl.ds(16 * j, 16)] * v_buf[e, pl.ds(16 * j, 16)]
                     for j in range(N_FEAT // 16)]
                accs.append(((p[0] + p[1]) + (p[2] + p[3]))
                            + ((p[4] + p[5]) + (p[6] + p[7])))
            m0 = jnp.where(msk8, step(accs[0], 8), step(accs[1], 8))
            m1 = jnp.where(msk8, step(accs[2], 8), step(accs[3], 8))
            n = jnp.where(msk4, step(m0, 4), step(m1, 4))
            full = step(step(n, 2), 1)
            scores = jnp.where(q == 0, jnp.zeros((16,), jnp.float32), scores)
            scores = jnp.where((lanes >> 2) == q, jnp.take(full, quad_pat), scores)
            out_buf[pl.ds((k >> 2) * 16, 16)] = scores
            return scores

        lax.fori_loop(0, CHUNK // 4, quad_body, jnp.zeros((16,), jnp.float32))

    def writeback(base, out_buf, sem):
        return pltpu.async_copy(out_buf, out_hbm.at[pl.ds(base0 + base, CHUNK)], sem)

    def reclaim(out_buf, sem):
        pltpu.make_async_copy(out_buf, out_hbm.at[pl.ds(base0, CHUNK)], sem).wait()

    cu, cv = gathers(0, u0, v0, sem_u0, sem_v0)
    cu.wait()
    cv.wait()

    def pair_body(i, carry):
        c0_base = (2 * i) * CHUNK
        c1_base = c0_base + CHUNK
        n0_base = jnp.minimum(c0_base + 2 * CHUNK, LAST_BASE)

        cu1, cv1 = gathers(c1_base, u1, v1, sem_u1, sem_v1)

        @pl.when(i > 0)
        def _():
            reclaim(out0, sem_o0)

        compute(u0, v0, out0)
        writeback(c0_base, out0, sem_o0)
        cu1.wait()
        cv1.wait()

        cu0, cv0 = gathers(n0_base, u0, v0, sem_u0, sem_v0)

        @pl.when(i > 0)
        def _():
            reclaim(out1, sem_o1)

        compute(u1, v1, out1)
        writeback(c1_base, out1, sem_o1)
        cu0.wait()
        cv0.wait()
        return carry

    lax.fori_loop(0, N_PAIRS, pair_body, 0)

    reclaim(out0, sem_o0)
    compute(u0, v0, out0)
    cp = writeback(LAST_BASE, out0, sem_o0)
    reclaim(out1, sem_o1)
    cp.wait()


def kernel(h, edge_index, r):
    hr = _scale_h_by_r(h, r)
    return _edge_scores(hr, h, edge_index.reshape(-1))
